# baseline (speedup 1.0000x reference)
import jax
import jax.numpy as jnp
from jax import lax
from jax.experimental import pallas as pl
from jax.experimental.pallas import tpu as pltpu
from jax.experimental.pallas import tpu_sc as plsc

NC = 26
NE = 182
EPAD = 192
AN = 32
B = 64
BS = 16
P = 16
NP = 14


def _adj_sc_body(src_hbm, dst_hbm, out_hbm, src_v, dst_v, dinv_v, abuf):
    cid = lax.axis_index("c")
    sid = lax.axis_index("s")

    @pl.when((cid == 0) & (sid == 0))
    def _():
        f32 = jnp.float32
        pltpu.sync_copy(src_hbm, src_v)
        pltpu.sync_copy(dst_hbm, dst_v)
        zeros = jnp.zeros((16,), f32)
        ones = jnp.full((16,), 1.0, f32)
        for k in range(AN * AN // 16):
            abuf[pl.ds(16 * k, 16)] = zeros
        for k in range(EPAD // 16):
            s_idx = src_v[pl.ds(16 * k, 16)]
            d_idx = dst_v[pl.ds(16 * k, 16)]
            plsc.addupdate_scatter(abuf, [s_idx * AN + d_idx], ones)
        lo = ones
        hi = ones
        for s in range(AN):
            lo = lo + abuf[pl.ds(AN * s, 16)]
            hi = hi + abuf[pl.ds(AN * s + 16, 16)]
        for t, deg in ((0, lo), (1, hi)):
            i = plsc.bitcast(deg, jnp.int32)
            i = 0x5F3759DF - (i >> 1)
            y = plsc.bitcast(i, f32)
            for _ in range(4):
                y = y * (1.5 - 0.5 * deg * y * y)
            dinv_v[pl.ds(16 * t, 16)] = y
        for k in range(AN * AN // 16):
            abuf[pl.ds(16 * k, 16)] = zeros
        for k in range(EPAD // 16):
            s_idx = src_v[pl.ds(16 * k, 16)]
            d_idx = dst_v[pl.ds(16 * k, 16)]
            gs = plsc.load_gather(dinv_v, [s_idx])
            gd = plsc.load_gather(dinv_v, [d_idx])
            plsc.addupdate_scatter(abuf, [d_idx * AN + s_idx], gs * gd)
        iota = lax.iota(jnp.int32, 16)
        for t in range(2):
            dv = dinv_v[pl.ds(16 * t, 16)]
            plsc.store_scatter(abuf, [(iota + 16 * t) * (AN + 1)], dv * dv)
        pltpu.sync_copy(abuf, out_hbm)


def _adj_sc(srcp, dstp):
    return pl.kernel(
        _adj_sc_body,
        out_type=jax.ShapeDtypeStruct((AN * AN,), jnp.float32),
        mesh=plsc.VectorSubcoreMesh(core_axis_name="c", subcore_axis_name="s"),
        compiler_params=pltpu.CompilerParams(needs_layout_passes=False),
        scratch_types=[
            pltpu.VMEM((EPAD,), jnp.int32),
            pltpu.VMEM((EPAD,), jnp.int32),
            pltpu.VMEM((AN,), jnp.float32),
            pltpu.VMEM((AN * AN,), jnp.float32),
        ],
    )(srcp, dstp)


def _fused_kernel(x_ref, wpat_ref, w1_ref, b1_ref, w2_ref, b2_ref,
                  wp_ref, bp_ref, emb_ref, wg1_ref, bg1_ref,
                  wg2_ref, bg2_ref, adj_ref, o_ref, m_ref):
    f32 = jnp.float32
    b = pl.program_id(0)

    w_id = lax.broadcasted_iota(jnp.int32, (NP * P, 3 * P), 0)
    k_id = lax.broadcasted_iota(jnp.int32, (NP * P, 3 * P), 1)
    combs = [((k_id % 3 == c) & (w_id % P == k_id // 3)).astype(f32)
             for c in range(3)]

    for bi in range(BS):
        xr = x_ref[bi]
        s = xr[:, 0:P, :]
        for ph in range(1, NP):
            s = s + xr[:, ph * P:(ph + 1) * P, :]
        row = jnp.zeros((P, 3 * P), f32)
        for c in range(3):
            row = row + jnp.dot(s[c], combs[c], preferred_element_type=f32)
        flat = jnp.concatenate([row[i:i + 1, :] for i in range(P)], axis=1)
        m_ref[pl.ds(b * BS + bi, 1), :] = flat * (1.0 / (NP * NP))

    @pl.when(b == B // BS - 1)
    def _head():
        m = m_ref[...]
        feat = jnp.dot(m, wpat_ref[...], preferred_element_type=f32)

        t1 = jnp.maximum(jnp.dot(feat, w1_ref[...], preferred_element_type=f32)
                         + b1_ref[0], 0.0)
        cnn_logits = jnp.dot(t1, w2_ref[...], preferred_element_type=f32) + b2_ref[0]

        proj = jnp.maximum(jnp.dot(feat, wp_ref[...], preferred_element_type=f32)
                           + bp_ref[0], 0.0)

        adj = adj_ref[...][0:NC, 0:NC]

        g1 = jnp.dot(emb_ref[...], wg1_ref[...], preferred_element_type=f32)
        h1 = jnp.maximum(jnp.dot(adj, g1, preferred_element_type=f32)
                         + bg1_ref[0], 0.0)
        g2 = jnp.dot(h1, wg2_ref[...], preferred_element_type=f32)
        h2 = jnp.dot(adj, g2, preferred_element_type=f32) + bg2_ref[0]

        gcn_logits = lax.dot_general(proj, h2, (((1,), (1,)), ((), ())),
                                     preferred_element_type=f32)
        o_ref[...] = cnn_logits + gcn_logits


def kernel(x, W_patch, W_i2c1, b_i2c1, W_i2c2, b_i2c2, W_proj, b_proj,
           class_emb, W_g1, b_g1, W_g2, b_g2, edge_index):
    srcp = jnp.pad(edge_index[0].astype(jnp.int32), (0, EPAD - NE),
                   constant_values=AN - 1)
    dstp = jnp.pad(edge_index[1].astype(jnp.int32), (0, EPAD - NE),
                   constant_values=AN - 1)
    adjm = _adj_sc(srcp, dstp).reshape(AN, AN)

    def const(shape):
        n = len(shape)
        return pl.BlockSpec(shape, lambda b, _n=n: (0,) * _n)

    out = pl.pallas_call(
        _fused_kernel,
        grid=(B // BS,),
        in_specs=[
            pl.BlockSpec((BS, 3, NP * P, NP * P), lambda b: (b, 0, 0, 0)),
            const((768, 2048)),
            const((2048, 1024)), const((1, 1024)),
            const((1024, NC)), const((1, NC)),
            const((2048, 1024)), const((1, 1024)),
            const((NC, 1024)),
            const((1024, 512)), const((1, 512)),
            const((512, 1024)), const((1, 1024)),
            const((AN, AN)),
        ],
        out_specs=const((B, NC)),
        out_shape=jax.ShapeDtypeStruct((B, NC), jnp.float32),
        scratch_shapes=[pltpu.VMEM((B, 3 * P * P), jnp.float32)],
    )(x, W_patch,
      W_i2c1, b_i2c1.reshape(1, -1), W_i2c2, b_i2c2.reshape(1, -1),
      W_proj, b_proj.reshape(1, -1), class_emb,
      W_g1, b_g1.reshape(1, -1), W_g2, b_g2.reshape(1, -1),
      adjm)
    return out


# speedup vs baseline: 1.0530x; 1.0530x over previous
import jax
import jax.numpy as jnp
from jax import lax
from jax.experimental import pallas as pl
from jax.experimental.pallas import tpu as pltpu
from jax.experimental.pallas import tpu_sc as plsc

NC = 26
NE = 182
EPAD = 192
AN = 32
B = 64
BS = 16
P = 16
NP = 14


def _adj_sc_body(src_hbm, dst_hbm, out_hbm, src_v, dst_v, dinv_v, abuf):
    cid = lax.axis_index("c")
    sid = lax.axis_index("s")

    @pl.when((cid == 0) & (sid == 0))
    def _():
        f32 = jnp.float32
        pltpu.sync_copy(src_hbm, src_v)
        pltpu.sync_copy(dst_hbm, dst_v)
        zeros = jnp.zeros((16,), f32)
        ones = jnp.full((16,), 1.0, f32)
        for k in range(AN * AN // 16):
            abuf[pl.ds(16 * k, 16)] = zeros
        for k in range(EPAD // 16):
            s_idx = src_v[pl.ds(16 * k, 16)]
            d_idx = dst_v[pl.ds(16 * k, 16)]
            plsc.addupdate_scatter(abuf, [s_idx * AN + d_idx], ones)
        lo = ones
        hi = ones
        for s in range(AN):
            lo = lo + abuf[pl.ds(AN * s, 16)]
            hi = hi + abuf[pl.ds(AN * s + 16, 16)]
        for t, deg in ((0, lo), (1, hi)):
            i = plsc.bitcast(deg, jnp.int32)
            i = 0x5F3759DF - (i >> 1)
            y = plsc.bitcast(i, f32)
            for _ in range(4):
                y = y * (1.5 - 0.5 * deg * y * y)
            dinv_v[pl.ds(16 * t, 16)] = y
        for k in range(AN * AN // 16):
            abuf[pl.ds(16 * k, 16)] = zeros
        for k in range(EPAD // 16):
            s_idx = src_v[pl.ds(16 * k, 16)]
            d_idx = dst_v[pl.ds(16 * k, 16)]
            gs = plsc.load_gather(dinv_v, [s_idx])
            gd = plsc.load_gather(dinv_v, [d_idx])
            plsc.addupdate_scatter(abuf, [d_idx * AN + s_idx], gs * gd)
        iota = lax.iota(jnp.int32, 16)
        for t in range(2):
            dv = dinv_v[pl.ds(16 * t, 16)]
            plsc.store_scatter(abuf, [(iota + 16 * t) * (AN + 1)], dv * dv)
        pltpu.sync_copy(abuf, out_hbm)


def _adj_sc(srcp, dstp):
    return pl.kernel(
        _adj_sc_body,
        out_type=jax.ShapeDtypeStruct((AN * AN,), jnp.float32),
        mesh=plsc.VectorSubcoreMesh(core_axis_name="c", subcore_axis_name="s"),
        compiler_params=pltpu.CompilerParams(needs_layout_passes=False),
        scratch_types=[
            pltpu.VMEM((EPAD,), jnp.int32),
            pltpu.VMEM((EPAD,), jnp.int32),
            pltpu.VMEM((AN,), jnp.float32),
            pltpu.VMEM((AN * AN,), jnp.float32),
        ],
    )(srcp, dstp)


def _fused_kernel(x_ref, wpat_ref, w1_ref, b1_ref, w2_ref, b2_ref,
                  wp_ref, bp_ref, emb_ref, wg1_ref,
                  cnn_ref, proj_ref, g1_ref, m_ref):
    f32 = jnp.float32
    b = pl.program_id(0)

    w_id = lax.broadcasted_iota(jnp.int32, (NP * P, 3 * P), 0)
    k_id = lax.broadcasted_iota(jnp.int32, (NP * P, 3 * P), 1)
    combs = [((k_id % 3 == c) & (w_id % P == k_id // 3)).astype(f32)
             for c in range(3)]

    for bi in range(BS):
        xr = x_ref[bi]
        s = xr[:, 0:P, :]
        for ph in range(1, NP):
            s = s + xr[:, ph * P:(ph + 1) * P, :]
        row = jnp.zeros((P, 3 * P), f32)
        for c in range(3):
            row = row + jnp.dot(s[c], combs[c], preferred_element_type=f32)
        flat = jnp.concatenate([row[i:i + 1, :] for i in range(P)], axis=1)
        m_ref[pl.ds(b * BS + bi, 1), :] = flat * (1.0 / (NP * NP))

    @pl.when(b == B // BS - 1)
    def _head():
        m = m_ref[...]
        feat = jnp.dot(m, wpat_ref[...], preferred_element_type=f32)

        t1 = jnp.maximum(jnp.dot(feat, w1_ref[...], preferred_element_type=f32)
                         + b1_ref[0], 0.0)
        cnn_ref[...] = (jnp.dot(t1, w2_ref[...], preferred_element_type=f32)
                        + b2_ref[0])
        proj_ref[...] = jnp.maximum(
            jnp.dot(feat, wp_ref[...], preferred_element_type=f32)
            + bp_ref[0], 0.0)
        g1_ref[...] = jnp.dot(emb_ref[...], wg1_ref[...],
                              preferred_element_type=f32)


def _gcn_kernel(adj_ref, g1_ref, bg1_ref, wg2_ref, bg2_ref,
                cnn_ref, proj_ref, o_ref):
    f32 = jnp.float32
    adj = adj_ref[...][0:NC, 0:NC]
    h1 = jnp.maximum(jnp.dot(adj, g1_ref[...], preferred_element_type=f32)
                     + bg1_ref[0], 0.0)
    g2 = jnp.dot(h1, wg2_ref[...], preferred_element_type=f32)
    h2 = jnp.dot(adj, g2, preferred_element_type=f32) + bg2_ref[0]
    gcn_logits = lax.dot_general(proj_ref[...], h2, (((1,), (1,)), ((), ())),
                                 preferred_element_type=f32)
    o_ref[...] = cnn_ref[...] + gcn_logits


def kernel(x, W_patch, W_i2c1, b_i2c1, W_i2c2, b_i2c2, W_proj, b_proj,
           class_emb, W_g1, b_g1, W_g2, b_g2, edge_index):
    srcp = jnp.pad(edge_index[0].astype(jnp.int32), (0, EPAD - NE),
                   constant_values=AN - 1)
    dstp = jnp.pad(edge_index[1].astype(jnp.int32), (0, EPAD - NE),
                   constant_values=AN - 1)
    adjm = _adj_sc(srcp, dstp).reshape(AN, AN)

    def const(shape):
        n = len(shape)
        return pl.BlockSpec(shape, lambda b, _n=n: (0,) * _n)

    cnn, proj, g1 = pl.pallas_call(
        _fused_kernel,
        grid=(B // BS,),
        in_specs=[
            pl.BlockSpec((BS, 3, NP * P, NP * P), lambda b: (b, 0, 0, 0)),
            const((768, 2048)),
            const((2048, 1024)), const((1, 1024)),
            const((1024, NC)), const((1, NC)),
            const((2048, 1024)), const((1, 1024)),
            const((NC, 1024)),
            const((1024, 512)),
        ],
        out_specs=[const((B, NC)), const((B, 1024)), const((NC, 512))],
        out_shape=[jax.ShapeDtypeStruct((B, NC), jnp.float32),
                   jax.ShapeDtypeStruct((B, 1024), jnp.float32),
                   jax.ShapeDtypeStruct((NC, 512), jnp.float32)],
        scratch_shapes=[pltpu.VMEM((B, 3 * P * P), jnp.float32)],
    )(x, W_patch,
      W_i2c1, b_i2c1.reshape(1, -1), W_i2c2, b_i2c2.reshape(1, -1),
      W_proj, b_proj.reshape(1, -1), class_emb, W_g1)

    out = pl.pallas_call(
        _gcn_kernel,
        out_shape=jax.ShapeDtypeStruct((B, NC), jnp.float32),
    )(adjm, g1, b_g1.reshape(1, -1), W_g2, b_g2.reshape(1, -1), cnn, proj)
    return out


# speedup vs baseline: 1.6393x; 1.5568x over previous
import jax
import jax.numpy as jnp
from jax import lax
from jax.experimental import pallas as pl
from jax.experimental.pallas import tpu as pltpu

NC = 26
ER = 7
B = 64
BS = 16
P = 16
NP = 14


def _fused_kernel(x_ref, wpat_ref, w1_ref, b1_ref, w2_ref, b2_ref,
                  wp_ref, bp_ref, emb_ref, wg1_ref, bg1_ref,
                  wg2_ref, bg2_ref, src_ref, dst_ref, o_ref, m_ref):
    f32 = jnp.float32
    b = pl.program_id(0)

    w_id = lax.broadcasted_iota(jnp.int32, (NP * P, 3 * P), 0)
    k_id = lax.broadcasted_iota(jnp.int32, (NP * P, 3 * P), 1)
    combs = [((k_id % 3 == c) & (w_id % P == k_id // 3)).astype(f32)
             for c in range(3)]

    for bi in range(BS):
        xr = x_ref[bi]
        s = xr[:, 0:P, :]
        for ph in range(1, NP):
            s = s + xr[:, ph * P:(ph + 1) * P, :]
        row = jnp.zeros((P, 3 * P), f32)
        for c in range(3):
            row = row + jnp.dot(s[c], combs[c], preferred_element_type=f32)
        flat = jnp.concatenate([row[i:i + 1, :] for i in range(P)], axis=1)
        m_ref[pl.ds(b * BS + bi, 1), :] = flat * (1.0 / (NP * NP))

    @pl.when(b == B // BS - 1)
    def _head():
        m = m_ref[...]
        feat = jnp.dot(m, wpat_ref[...], preferred_element_type=f32)

        t1 = jnp.maximum(jnp.dot(feat, w1_ref[...], preferred_element_type=f32)
                         + b1_ref[0], 0.0)
        cnn_logits = jnp.dot(t1, w2_ref[...], preferred_element_type=f32) + b2_ref[0]

        proj = jnp.maximum(jnp.dot(feat, wp_ref[...], preferred_element_type=f32)
                           + bp_ref[0], 0.0)

        src = src_ref[...]
        dst = dst_ref[...]
        node = lax.broadcasted_iota(jnp.int32, (ER, NC, NC), 2)
        oh_dst = (dst[:, :, None] == node).astype(f32)
        oh_src = (src[:, :, None] == node).astype(f32)
        deg = jnp.sum(oh_dst, axis=(0, 1)) + 1.0
        dinv = lax.rsqrt(deg)
        dinv_s = jnp.sum(oh_src * dinv[None, None, :], axis=2)
        dinv_d = jnp.sum(oh_dst * dinv[None, None, :], axis=2)
        norm = dinv_s * dinv_d
        lhs = (oh_dst * norm[:, :, None]).reshape(ER * NC, NC)
        rhs = oh_src.reshape(ER * NC, NC)
        adj = lax.dot_general(lhs, rhs, (((0,), (0,)), ((), ())),
                              preferred_element_type=f32)
        r_id = lax.broadcasted_iota(jnp.int32, (NC, NC), 0)
        c_id = lax.broadcasted_iota(jnp.int32, (NC, NC), 1)
        adj = adj + jnp.where(r_id == c_id, dinv * dinv, 0.0)

        g1 = jnp.dot(emb_ref[...], wg1_ref[...], preferred_element_type=f32)
        h1 = jnp.maximum(jnp.dot(adj, g1, preferred_element_type=f32)
                         + bg1_ref[0], 0.0)
        g2 = jnp.dot(h1, wg2_ref[...], preferred_element_type=f32)
        h2 = jnp.dot(adj, g2, preferred_element_type=f32) + bg2_ref[0]

        gcn_logits = lax.dot_general(proj, h2, (((1,), (1,)), ((), ())),
                                     preferred_element_type=f32)
        o_ref[...] = cnn_logits + gcn_logits


def kernel(x, W_patch, W_i2c1, b_i2c1, W_i2c2, b_i2c2, W_proj, b_proj,
           class_emb, W_g1, b_g1, W_g2, b_g2, edge_index):
    src8 = edge_index[0].astype(jnp.int32).reshape(ER, NC)
    dst8 = edge_index[1].astype(jnp.int32).reshape(ER, NC)

    def const(shape):
        n = len(shape)
        return pl.BlockSpec(shape, lambda b, _n=n: (0,) * _n)

    out = pl.pallas_call(
        _fused_kernel,
        grid=(B // BS,),
        in_specs=[
            pl.BlockSpec((BS, 3, NP * P, NP * P), lambda b: (b, 0, 0, 0)),
            const((768, 2048)),
            const((2048, 1024)), const((1, 1024)),
            const((1024, NC)), const((1, NC)),
            const((2048, 1024)), const((1, 1024)),
            const((NC, 1024)),
            const((1024, 512)), const((1, 512)),
            const((512, 1024)), const((1, 1024)),
            const((ER, NC)), const((ER, NC)),
        ],
        out_specs=const((B, NC)),
        out_shape=jax.ShapeDtypeStruct((B, NC), jnp.float32),
        scratch_shapes=[pltpu.VMEM((B, 3 * P * P), jnp.float32)],
    )(x, W_patch,
      W_i2c1, b_i2c1.reshape(1, -1), W_i2c2, b_i2c2.reshape(1, -1),
      W_proj, b_proj.reshape(1, -1), class_emb,
      W_g1, b_g1.reshape(1, -1), W_g2, b_g2.reshape(1, -1),
      src8, dst8)
    return out
